# Initial kernel scaffold; baseline (speedup 1.0000x reference)
import jax
import jax.numpy as jnp
from jax.experimental import pallas as pl


def kernel(x, y, projections):
    raise NotImplementedError("write your pallas kernel here")


# speedup vs baseline: 22.5492x; 22.5492x over previous
import jax
import jax.numpy as jnp
from jax.experimental import pallas as pl


def _bitonic_sort_rows(key, vals, C, N):
    lane = jax.lax.broadcasted_iota(jnp.int32, (C, N), 1)
    logn = N.bit_length() - 1
    for kk in range(1, logn + 1):
        desc = ((lane >> kk) & 1) == 1
        for j in range(kk - 1, -1, -1):
            s = 1 << j
            lo = (lane & s) == 0

            def partner(a):
                return jnp.where(lo, jnp.roll(a, -s, axis=1),
                                 jnp.roll(a, s, axis=1))

            pk = partner(key)
            take = ((lo & (key > pk)) | (~lo & (key < pk))) != desc
            key = jnp.where(take, pk, key)
            vals = [jnp.where(take, partner(v), v) for v in vals]
    return key, vals


def _body(x_ref, y_ref, p_ref, o_ref):
    b = pl.program_id(0)
    lc = pl.program_id(1)
    C, N = p_ref.shape[1], x_ref.shape[2]
    D = x_ref.shape[1]

    p = p_ref[0]
    xt = x_ref[0]
    yt = y_ref[0]

    xk = jnp.zeros((C, N), jnp.float32)
    yk = jnp.zeros((C, N), jnp.float32)
    for d in range(D):
        xk = xk + p[:, d:d + 1] * xt[d:d + 1, :]
        yk = yk + p[:, d:d + 1] * yt[d:d + 1, :]

    xv = [jnp.broadcast_to(xt[d:d + 1, :], (C, N)) for d in range(D)]
    yv = [jnp.broadcast_to(yt[d:d + 1, :], (C, N)) for d in range(D)]

    _, xv = _bitonic_sort_rows(xk, xv, C, N)
    _, yv = _bitonic_sort_rows(yk, yv, C, N)

    acc = jnp.float32(0.0)
    for d in range(D):
        diff = xv[d] - yv[d]
        acc = acc + jnp.sum(diff * diff)

    @pl.when((b == 0) & (lc == 0))
    def _():
        o_ref[...] = jnp.zeros((1, 1), jnp.float32)

    o_ref[...] += jnp.reshape(acc, (1, 1))


def _pairing_loss_sum(x, y, projections, interpret=False):
    B, N, D = x.shape
    L = projections.shape[1]
    C = 16 if L % 16 == 0 else L
    x_t = jnp.transpose(x, (0, 2, 1))
    y_t = jnp.transpose(y, (0, 2, 1))
    out = pl.pallas_call(
        _body,
        grid=(B, L // C),
        in_specs=[
            pl.BlockSpec((1, D, N), lambda b, l: (b, 0, 0)),
            pl.BlockSpec((1, D, N), lambda b, l: (b, 0, 0)),
            pl.BlockSpec((1, C, D), lambda b, l: (b, l, 0)),
        ],
        out_specs=pl.BlockSpec((1, 1), lambda b, l: (0, 0)),
        out_shape=jax.ShapeDtypeStruct((1, 1), jnp.float32),
        interpret=interpret,
    )(x_t, y_t, projections)
    return out[0, 0]


def kernel(x, y, projections):
    B, N, D = x.shape
    L = projections.shape[1]
    total = _pairing_loss_sum(x, y, projections)
    return (total / jnp.float32(B * L * N * D)).astype(jnp.float32)


# speedup vs baseline: 71.0053x; 3.1489x over previous
import functools

import jax
import jax.numpy as jnp
from jax import lax
from jax.experimental import pallas as pl
from jax.experimental.pallas import tpu as pltpu
from jax.experimental.pallas import tpu_sc as plsc

_C = 32


def _sort_body(x_ref, y_ref, p_ref, xi_ref, yi_ref):
    C, N = xi_ref.shape[1], xi_ref.shape[2]
    D = x_ref.shape[1]
    logn = N.bit_length() - 1
    idx_mask = jnp.int32(N - 1)
    key_mask = jnp.int32(-N)
    flip = jnp.int32(0x7FFFFFFF)

    p = p_ref[0]
    lane = lax.broadcasted_iota(jnp.int32, (C, N), 1)

    def make_key(t_ref):
        t = t_ref[0]
        k = jnp.zeros((C, N), jnp.float32)
        for d in range(D):
            k = k + p[:, d:d + 1] * t[d:d + 1, :]
        bi = lax.bitcast_convert_type(k, jnp.int32)
        bi = (bi & key_mask) | lane
        return jnp.where(bi < 0, bi ^ flip, bi)

    xu = make_key(x_ref)
    yu = make_key(y_ref)

    for kk in range(1, logn + 1):
        for j in range(kk - 1, -1, -1):
            s = 1 << j
            lo = (lane & s) == 0
            sel_min = (((lane >> j) ^ (lane >> kk)) & 1) == 0

            def cmpex(a):
                pa = jnp.where(lo, jnp.roll(a, -s, axis=1),
                               jnp.roll(a, s, axis=1))
                return jnp.where(sel_min, jnp.minimum(a, pa),
                                 jnp.maximum(a, pa))

            xu = cmpex(xu)
            yu = cmpex(yu)

    def extract(u):
        return jnp.where(u >= 0, u, ~u) & idx_mask

    xi_ref[0] = extract(xu)
    yi_ref[0] = extract(yu)


def _rank_indices(x_t, y_t, projections, interpret=False):
    B, D, N = x_t.shape
    L = projections.shape[1]
    C = _C if L % _C == 0 else L
    return pl.pallas_call(
        _sort_body,
        grid=(B, L // C),
        in_specs=[
            pl.BlockSpec((1, D, N), lambda b, l: (b, 0, 0)),
            pl.BlockSpec((1, D, N), lambda b, l: (b, 0, 0)),
            pl.BlockSpec((1, C, D), lambda b, l: (b, l, 0)),
        ],
        out_specs=[
            pl.BlockSpec((1, C, N), lambda b, l: (b, l, 0)),
            pl.BlockSpec((1, C, N), lambda b, l: (b, l, 0)),
        ],
        out_shape=[
            jax.ShapeDtypeStruct((B, L, N), jnp.int32),
            jax.ShapeDtypeStruct((B, L, N), jnp.int32),
        ],
        interpret=interpret,
    )(x_t, y_t, projections)


def _sc_pair_reduce(x_t, y_t, xidx, yidx):
    B, D, N = x_t.shape
    L = xidx.shape[1]
    info = plsc.get_sparse_core_info()
    NC, NS, LN = info.num_cores, info.num_subcores, info.num_lanes
    NW = NC * NS
    R = (B * L) // NW
    WPB = L // R

    @functools.partial(
        pl.kernel,
        out_type=jax.ShapeDtypeStruct((NW * LN,), jnp.float32),
        mesh=plsc.VectorSubcoreMesh(core_axis_name="c", subcore_axis_name="s"),
        compiler_params=pltpu.CompilerParams(needs_layout_passes=False),
        scratch_types=(
            [pltpu.VMEM((N,), jnp.float32) for _ in range(2 * D)]
            + [
                pltpu.VMEM((N,), jnp.int32),
                pltpu.VMEM((N,), jnp.int32),
                pltpu.VMEM((LN,), jnp.float32),
            ]
        ),
    )
    def k(x_h, y_h, xi_h, yi_h, out_h, *scratch):
        tabs = scratch[:2 * D]
        xi_v, yi_v, acc_v = scratch[2 * D:]
        wid = lax.axis_index("s") * NC + lax.axis_index("c")
        b = wid // WPB
        l0 = (wid % WPB) * R
        for d in range(D):
            pltpu.sync_copy(x_h.at[pl.ds((b * D + d) * N, N)], tabs[d])
            pltpu.sync_copy(y_h.at[pl.ds((b * D + d) * N, N)], tabs[D + d])

        def row(i, acc):
            l = l0 + i
            pltpu.sync_copy(xi_h.at[pl.ds((b * L + l) * N, N)], xi_v)
            pltpu.sync_copy(yi_h.at[pl.ds((b * L + l) * N, N)], yi_v)

            def chunk(c, a):
                xi = xi_v[pl.ds(c * LN, LN)]
                yi = yi_v[pl.ds(c * LN, LN)]
                for d in range(D):
                    xg = plsc.load_gather(tabs[d], [xi])
                    yg = plsc.load_gather(tabs[D + d], [yi])
                    df = xg - yg
                    a = a + df * df
                return a

            return lax.fori_loop(0, N // LN, chunk, acc)

        acc = lax.fori_loop(0, R, row, jnp.zeros((LN,), jnp.float32))
        acc_v[...] = acc
        pltpu.sync_copy(acc_v, out_h.at[pl.ds(wid * LN, LN)])

    return k(x_t.reshape(-1), y_t.reshape(-1),
             xidx.reshape(-1), yidx.reshape(-1))


def kernel(x, y, projections):
    B, N, D = x.shape
    L = projections.shape[1]
    x_t = jnp.transpose(x, (0, 2, 1))
    y_t = jnp.transpose(y, (0, 2, 1))
    xidx, yidx = _rank_indices(x_t, y_t, projections)
    partials = _sc_pair_reduce(x_t, y_t, xidx, yidx)
    return (jnp.sum(partials) / jnp.float32(B * L * N * D)).astype(jnp.float32)


# speedup vs baseline: 76.7903x; 1.0815x over previous
import functools

import jax
import jax.numpy as jnp
from jax import lax
from jax.experimental import pallas as pl
from jax.experimental.pallas import tpu as pltpu
from jax.experimental.pallas import tpu_sc as plsc

_C = 64


def _sort_body(x_ref, y_ref, p_ref, xi_ref, yi_ref):
    C, N = xi_ref.shape[1], xi_ref.shape[2]
    D = x_ref.shape[1]
    logn = N.bit_length() - 1
    idx_mask = jnp.int32(N - 1)
    key_mask = jnp.int32(-N)
    flip = jnp.int32(0x7FFFFFFF)

    p = p_ref[0]
    lane = lax.broadcasted_iota(jnp.int32, (C, N), 1)

    def make_key(t_ref):
        t = t_ref[0]
        k = jnp.zeros((C, N), jnp.float32)
        for d in range(D):
            k = k + p[:, d:d + 1] * t[d:d + 1, :]
        bi = lax.bitcast_convert_type(k, jnp.int32)
        bi = (bi & key_mask) | lane
        return jnp.where(bi < 0, bi ^ flip, bi)

    xu = make_key(x_ref)
    yu = make_key(y_ref)

    lane1 = lax.broadcasted_iota(jnp.int32, (1, N), 1)
    for kk in range(1, logn + 1):
        for j in range(kk - 1, -1, -1):
            s = 1 << j
            lo = (lane1 & s) == 0
            sel_min = (((lane1 >> j) ^ (lane1 >> kk)) & 1) == 0

            def cmpex(a):
                pa = jnp.where(lo, jnp.roll(a, -s, axis=1),
                               jnp.roll(a, s, axis=1))
                return jnp.where(sel_min, jnp.minimum(a, pa),
                                 jnp.maximum(a, pa))

            xu = cmpex(xu)
            yu = cmpex(yu)

    def extract(u):
        return jnp.where(u >= 0, u, ~u) & idx_mask

    xi_ref[0] = extract(xu)
    yi_ref[0] = extract(yu)


def _rank_indices(x_t, y_t, projections, interpret=False):
    B, D, N = x_t.shape
    L = projections.shape[1]
    C = _C if L % _C == 0 else L
    return pl.pallas_call(
        _sort_body,
        grid=(B, L // C),
        in_specs=[
            pl.BlockSpec((1, D, N), lambda b, l: (b, 0, 0)),
            pl.BlockSpec((1, D, N), lambda b, l: (b, 0, 0)),
            pl.BlockSpec((1, C, D), lambda b, l: (b, l, 0)),
        ],
        out_specs=[
            pl.BlockSpec((1, C, N), lambda b, l: (b, l, 0)),
            pl.BlockSpec((1, C, N), lambda b, l: (b, l, 0)),
        ],
        out_shape=[
            jax.ShapeDtypeStruct((B, L, N), jnp.int32),
            jax.ShapeDtypeStruct((B, L, N), jnp.int32),
        ],
        interpret=interpret,
    )(x_t, y_t, projections)


def _sc_pair_reduce(x_t, y_t, xidx, yidx):
    B, D, N = x_t.shape
    L = xidx.shape[1]
    info = plsc.get_sparse_core_info()
    NC, NS, LN = info.num_cores, info.num_subcores, info.num_lanes
    NW = NC * NS
    R = (B * L) // NW
    WPB = L // R

    @functools.partial(
        pl.kernel,
        out_type=jax.ShapeDtypeStruct((NW * LN,), jnp.float32),
        mesh=plsc.VectorSubcoreMesh(core_axis_name="c", subcore_axis_name="s"),
        compiler_params=pltpu.CompilerParams(needs_layout_passes=False),
        scratch_types=(
            [pltpu.VMEM((N,), jnp.float32) for _ in range(2 * D)]
            + [
                pltpu.VMEM((N,), jnp.int32),
                pltpu.VMEM((N,), jnp.int32),
                pltpu.VMEM((LN,), jnp.float32),
            ]
        ),
    )
    def k(x_h, y_h, xi_h, yi_h, out_h, *scratch):
        tabs = scratch[:2 * D]
        xi_v, yi_v, acc_v = scratch[2 * D:]
        wid = lax.axis_index("s") * NC + lax.axis_index("c")
        b = wid // WPB
        l0 = (wid % WPB) * R
        for d in range(D):
            pltpu.sync_copy(x_h.at[pl.ds((b * D + d) * N, N)], tabs[d])
            pltpu.sync_copy(y_h.at[pl.ds((b * D + d) * N, N)], tabs[D + d])

        def row(i, acc):
            l = l0 + i
            pltpu.sync_copy(xi_h.at[pl.ds((b * L + l) * N, N)], xi_v)
            pltpu.sync_copy(yi_h.at[pl.ds((b * L + l) * N, N)], yi_v)

            def chunk(c, a):
                xi = xi_v[pl.ds(c * LN, LN)]
                yi = yi_v[pl.ds(c * LN, LN)]
                for d in range(D):
                    xg = plsc.load_gather(tabs[d], [xi])
                    yg = plsc.load_gather(tabs[D + d], [yi])
                    df = xg - yg
                    a = a + df * df
                return a

            return lax.fori_loop(0, N // LN, chunk, acc)

        acc = lax.fori_loop(0, R, row, jnp.zeros((LN,), jnp.float32))
        acc_v[...] = acc
        pltpu.sync_copy(acc_v, out_h.at[pl.ds(wid * LN, LN)])

    return k(x_t.reshape(-1), y_t.reshape(-1),
             xidx.reshape(-1), yidx.reshape(-1))


def kernel(x, y, projections):
    B, N, D = x.shape
    L = projections.shape[1]
    x_t = jnp.transpose(x, (0, 2, 1))
    y_t = jnp.transpose(y, (0, 2, 1))
    xidx, yidx = _rank_indices(x_t, y_t, projections)
    partials = _sc_pair_reduce(x_t, y_t, xidx, yidx)
    return (jnp.sum(partials) / jnp.float32(B * L * N * D)).astype(jnp.float32)


# speedup vs baseline: 78.3365x; 1.0201x over previous
import functools

import jax
import jax.numpy as jnp
from jax import lax
from jax.experimental import pallas as pl
from jax.experimental.pallas import tpu as pltpu
from jax.experimental.pallas import tpu_sc as plsc

_C = 64


def _sort_body(x_ref, y_ref, p_ref, xi_ref, yi_ref):
    C, N = xi_ref.shape[1], xi_ref.shape[2]
    D = x_ref.shape[1]
    logn = N.bit_length() - 1
    idx_mask = jnp.int32(N - 1)
    key_mask = jnp.int32(-N)
    flip = jnp.int32(0x7FFFFFFF)

    p = p_ref[0]
    lane = lax.broadcasted_iota(jnp.int32, (C, N), 1)

    def make_key(t_ref):
        t = t_ref[0]
        k = jnp.zeros((C, N), jnp.float32)
        for d in range(D):
            k = k + p[:, d:d + 1] * t[d:d + 1, :]
        bi = lax.bitcast_convert_type(k, jnp.int32)
        bi = (bi & key_mask) | lane
        return jnp.where(bi < 0, bi ^ flip, bi)

    xu = make_key(x_ref)
    yu = make_key(y_ref)

    lane1 = lax.broadcasted_iota(jnp.int32, (1, N), 1)
    for kk in range(1, logn + 1):
        for j in range(kk - 1, -1, -1):
            s = 1 << j
            lo = (lane1 & s) == 0
            sel_min = (((lane1 >> j) ^ (lane1 >> kk)) & 1) == 0

            def cmpex(a):
                pa = jnp.where(lo, jnp.roll(a, -s, axis=1),
                               jnp.roll(a, s, axis=1))
                return jnp.where(sel_min, jnp.minimum(a, pa),
                                 jnp.maximum(a, pa))

            xu = cmpex(xu)
            yu = cmpex(yu)

    def extract(u):
        return jnp.where(u >= 0, u, ~u) & idx_mask

    xi_ref[0] = extract(xu)
    yi_ref[0] = extract(yu)


def _rank_indices(x_t, y_t, projections, interpret=False):
    B, D, N = x_t.shape
    L = projections.shape[1]
    C = _C if L % _C == 0 else L
    return pl.pallas_call(
        _sort_body,
        grid=(B, L // C),
        in_specs=[
            pl.BlockSpec((1, D, N), lambda b, l: (b, 0, 0)),
            pl.BlockSpec((1, D, N), lambda b, l: (b, 0, 0)),
            pl.BlockSpec((1, C, D), lambda b, l: (b, l, 0)),
        ],
        out_specs=[
            pl.BlockSpec((1, C, N), lambda b, l: (b, l, 0)),
            pl.BlockSpec((1, C, N), lambda b, l: (b, l, 0)),
        ],
        out_shape=[
            jax.ShapeDtypeStruct((B, L, N), jnp.int32),
            jax.ShapeDtypeStruct((B, L, N), jnp.int32),
        ],
        interpret=interpret,
    )(x_t, y_t, projections)


def _sc_pair_reduce(x_t, y_t, xidx, yidx):
    B, D, N = x_t.shape
    L = xidx.shape[1]
    info = plsc.get_sparse_core_info()
    NC, NS, LN = info.num_cores, info.num_subcores, info.num_lanes
    NW = NC * NS
    R = (B * L) // NW
    WPB = L // R
    RB = min(8, R)

    @functools.partial(
        pl.kernel,
        out_type=jax.ShapeDtypeStruct((NW * LN,), jnp.float32),
        mesh=plsc.VectorSubcoreMesh(core_axis_name="c", subcore_axis_name="s"),
        compiler_params=pltpu.CompilerParams(needs_layout_passes=False),
        scratch_types=(
            [pltpu.VMEM((N,), jnp.float32) for _ in range(2 * D)]
            + [
                pltpu.VMEM((RB * N,), jnp.int32),
                pltpu.VMEM((RB * N,), jnp.int32),
                pltpu.VMEM((LN,), jnp.float32),
            ]
        ),
    )
    def k(x_h, y_h, xi_h, yi_h, out_h, *scratch):
        tabs = scratch[:2 * D]
        xi_v, yi_v, acc_v = scratch[2 * D:]
        wid = lax.axis_index("s") * NC + lax.axis_index("c")
        b = wid // WPB
        l0 = (wid % WPB) * R
        for d in range(D):
            pltpu.sync_copy(x_h.at[pl.ds((b * D + d) * N, N)], tabs[d])
            pltpu.sync_copy(y_h.at[pl.ds((b * D + d) * N, N)], tabs[D + d])

        def rowblk(i, acc):
            base = (b * L + l0) * N + i * (RB * N)
            pltpu.sync_copy(xi_h.at[pl.ds(base, RB * N)], xi_v)
            pltpu.sync_copy(yi_h.at[pl.ds(base, RB * N)], yi_v)

            def chunk(c, a):
                xi = xi_v[pl.ds(c * LN, LN)]
                yi = yi_v[pl.ds(c * LN, LN)]
                for d in range(D):
                    xg = plsc.load_gather(tabs[d], [xi])
                    yg = plsc.load_gather(tabs[D + d], [yi])
                    df = xg - yg
                    a = a + df * df
                return a

            return lax.fori_loop(0, (RB * N) // LN, chunk, acc)

        acc = lax.fori_loop(0, R // RB, rowblk, jnp.zeros((LN,), jnp.float32))
        acc_v[...] = acc
        pltpu.sync_copy(acc_v, out_h.at[pl.ds(wid * LN, LN)])

    return k(x_t.reshape(-1), y_t.reshape(-1),
             xidx.reshape(-1), yidx.reshape(-1))


def kernel(x, y, projections):
    B, N, D = x.shape
    L = projections.shape[1]
    x_t = jnp.transpose(x, (0, 2, 1))
    y_t = jnp.transpose(y, (0, 2, 1))
    xidx, yidx = _rank_indices(x_t, y_t, projections)
    partials = _sc_pair_reduce(x_t, y_t, xidx, yidx)
    return (jnp.sum(partials) / jnp.float32(B * L * N * D)).astype(jnp.float32)
